# baseline (speedup 1.0000x reference)
import functools

import jax
import jax.numpy as jnp
from jax import lax
from jax.experimental import pallas as pl
from jax.experimental.pallas import tpu as pltpu
from jax.experimental.pallas import tpu_sc as plsc

_Q = 1024
_R = 8
_L = 512
_D = 147
_TEMP = 5.0
_ITERS = 10


_GPP = 4


def _soft_kmeans_body(x_ref, centers_ref, labels_ref):
    xs = [x_ref[g] for g in range(_GPP)]

    def dist(x, c):
        xc = lax.dot_general(
            x, c, (((1,), (1,)), ((), ())),
            preferred_element_type=jnp.float32,
            precision=lax.Precision.DEFAULT,
        )
        c2 = jnp.sum(c * c, axis=1)[None, :]
        x2 = jnp.sum(x * x, axis=1)[:, None]
        return x2 - 2.0 * xc + c2

    def one_step(x, c):
        d = dist(x, c)
        z = -_TEMP * d
        z = z - jnp.max(z, axis=1, keepdims=True)
        e = jnp.exp(z)
        p = e / jnp.sum(e, axis=1, keepdims=True)
        w = p / (jnp.sum(p, axis=0, keepdims=True) + 1e-9)
        return lax.dot_general(
            w, x, (((0,), (0,)), ((), ())),
            preferred_element_type=jnp.float32,
            precision=lax.Precision.DEFAULT,
        )

    def step(_, cs):
        return tuple(one_step(x, c) for x, c in zip(xs, cs))

    cs = lax.fori_loop(0, _ITERS, step, tuple(x[:_L, :] for x in xs))

    li = lax.broadcasted_iota(jnp.int32, (_Q, _L), 1)
    for g, (x, c) in enumerate(zip(xs, cs)):
        d = dist(x, c)
        m = jnp.min(d, axis=1, keepdims=True)
        lab = jnp.min(jnp.where(d == m, li, _L), axis=1, keepdims=True)
        labels_ref[g] = lab + (pl.program_id(0) * _GPP + g) * _L
        centers_ref[g] = c


def _run_soft_kmeans(xr):
    return pl.pallas_call(
        _soft_kmeans_body,
        grid=(_R // _GPP,),
        in_specs=[pl.BlockSpec((_GPP, _Q, _D), lambda r: (r, 0, 0))],
        out_specs=[
            pl.BlockSpec((_GPP, _L, _D), lambda r: (r, 0, 0)),
            pl.BlockSpec((_GPP, _Q, 1), lambda r: (r, 0, 0)),
        ],
        out_shape=[
            jax.ShapeDtypeStruct((_R, _L, _D), jnp.float32),
            jax.ShapeDtypeStruct((_R, _Q, 1), jnp.int32),
        ],
        compiler_params=pltpu.CompilerParams(
            dimension_semantics=("parallel",),
        ),
    )(xr)


_DP = 256


def _sc_gather(table, idx):
    info = plsc.get_sparse_core_info()
    nc, ns = info.num_cores, info.num_subcores
    nw = nc * ns
    b = _R * _Q
    b_per_w = b // nw
    mesh = plsc.VectorSubcoreMesh(core_axis_name="c", subcore_axis_name="s")

    @functools.partial(
        pl.kernel, mesh=mesh,
        out_type=jax.ShapeDtypeStruct((b, _DP), jnp.float32),
        scratch_types=[
            pltpu.VMEM((b_per_w,), jnp.int32),
            pltpu.VMEM((b_per_w, _DP), jnp.float32),
            pltpu.SemaphoreType.DMA,
        ],
    )
    def gather_k(table_hbm, idx_hbm, out_hbm, idx_v, rows_v, sem):
        wid = lax.axis_index("s") * nc + lax.axis_index("c")
        base = wid * b_per_w
        pltpu.sync_copy(idx_hbm.at[pl.ds(base, b_per_w)], idx_v)
        pltpu.async_copy(table_hbm.at[idx_v], rows_v, sem).wait()
        pltpu.sync_copy(rows_v, out_hbm.at[pl.ds(base, b_per_w)])

    return gather_k(table, idx)


def kernel(x):
    B, T, F = x.shape
    xr = x.reshape(_R, _Q, _D)
    centers, labels = _run_soft_kmeans(xr)
    table = jnp.pad(centers, ((0, 0), (0, 0), (0, _DP - _D)))
    table = table.reshape(_R * _L, _DP)
    idx = labels.reshape(_R * _Q)
    rec = _sc_gather(table, idx)
    return rec[:, :_D].reshape(B, T, F)


# speedup vs baseline: 1.0087x; 1.0087x over previous
import functools

import jax
import jax.numpy as jnp
from jax import lax
from jax.experimental import pallas as pl
from jax.experimental.pallas import tpu as pltpu
from jax.experimental.pallas import tpu_sc as plsc

_Q = 1024
_R = 8
_L = 512
_D = 147
_TEMP = 5.0
_ITERS = 10


_GPP = 2

_DP = 256


def _soft_kmeans_body(x_ref, centers_ref, labels_ref):
    xs = [x_ref[g] for g in range(_GPP)]

    def dist(x, c):
        xc = lax.dot_general(
            x, c, (((1,), (1,)), ((), ())),
            preferred_element_type=jnp.float32,
            precision=lax.Precision.DEFAULT,
        )
        c2 = jnp.sum(c * c, axis=1)[None, :]
        x2 = jnp.sum(x * x, axis=1)[:, None]
        return x2 - 2.0 * xc + c2

    def one_step(x, c):
        d = dist(x, c)
        z = -_TEMP * d
        z = z - jnp.max(z, axis=1, keepdims=True)
        e = jnp.exp(z)
        p = e / jnp.sum(e, axis=1, keepdims=True)
        w = p / (jnp.sum(p, axis=0, keepdims=True) + 1e-9)
        return lax.dot_general(
            w, x, (((0,), (0,)), ((), ())),
            preferred_element_type=jnp.float32,
            precision=lax.Precision.DEFAULT,
        )

    def step(_, cs):
        return tuple(one_step(x, c) for x, c in zip(xs, cs))

    cs = lax.fori_loop(0, _ITERS, step, tuple(x[:_L, :] for x in xs))

    li = lax.broadcasted_iota(jnp.int32, (_Q, _L), 1)
    for g, (x, c) in enumerate(zip(xs, cs)):
        d = dist(x, c)
        m = jnp.min(d, axis=1, keepdims=True)
        lab = jnp.min(jnp.where(d == m, li, _L), axis=1, keepdims=True)
        labels_ref[g] = lab + (pl.program_id(0) * _GPP + g) * _L
        centers_ref[g, :, : _D] = c


def _run_soft_kmeans(xr):
    return pl.pallas_call(
        _soft_kmeans_body,
        grid=(_R // _GPP,),
        in_specs=[pl.BlockSpec((_GPP, _Q, _D), lambda r: (r, 0, 0))],
        out_specs=[
            pl.BlockSpec((_GPP, _L, _DP), lambda r: (r, 0, 0)),
            pl.BlockSpec((_GPP, _Q, 1), lambda r: (r, 0, 0)),
        ],
        out_shape=[
            jax.ShapeDtypeStruct((_R, _L, _DP), jnp.float32),
            jax.ShapeDtypeStruct((_R, _Q, 1), jnp.int32),
        ],
        compiler_params=pltpu.CompilerParams(
            dimension_semantics=("parallel",),
        ),
    )(xr)


def _sc_gather(table, idx):
    info = plsc.get_sparse_core_info()
    nc, ns = info.num_cores, info.num_subcores
    nw = nc * ns
    b = _R * _Q
    b_per_w = b // nw
    mesh = plsc.VectorSubcoreMesh(core_axis_name="c", subcore_axis_name="s")

    @functools.partial(
        pl.kernel, mesh=mesh,
        out_type=jax.ShapeDtypeStruct((b, _DP), jnp.float32),
        scratch_types=[
            pltpu.VMEM((b_per_w,), jnp.int32),
            pltpu.VMEM((b_per_w, _DP), jnp.float32),
            pltpu.SemaphoreType.DMA,
        ],
    )
    def gather_k(table_hbm, idx_hbm, out_hbm, idx_v, rows_v, sem):
        wid = lax.axis_index("s") * nc + lax.axis_index("c")
        base = wid * b_per_w
        pltpu.sync_copy(idx_hbm.at[pl.ds(base, b_per_w)], idx_v)
        pltpu.async_copy(table_hbm.at[idx_v], rows_v, sem).wait()
        pltpu.sync_copy(rows_v, out_hbm.at[pl.ds(base, b_per_w)])

    return gather_k(table, idx)


def kernel(x):
    B, T, F = x.shape
    xr = x.reshape(_R, _Q, _D)
    centers, labels = _run_soft_kmeans(xr)
    table = centers.reshape(_R * _L, _DP)
    idx = labels.reshape(_R * _Q)
    rec = _sc_gather(table, idx)
    return rec[:, :_D].reshape(B, T, F)


# speedup vs baseline: 1.0191x; 1.0103x over previous
import functools

import jax
import jax.numpy as jnp
from jax import lax
from jax.experimental import pallas as pl
from jax.experimental.pallas import tpu as pltpu
from jax.experimental.pallas import tpu_sc as plsc

_Q = 1024
_R = 8
_L = 512
_D = 147
_TEMP = 5.0
_ITERS = 10


_GPP = 2

_DP = 256


def _soft_kmeans_body(x_ref, centers_ref, labels_ref):
    xs = [x_ref[g] for g in range(_GPP)]
    x2s = [jnp.sum(x * x, axis=1)[:, None] for x in xs]

    def dist(x, x2, c):
        xc = lax.dot_general(
            x, c, (((1,), (1,)), ((), ())),
            preferred_element_type=jnp.float32,
            precision=lax.Precision.DEFAULT,
        )
        c2 = jnp.sum(c * c, axis=1)[None, :]
        return x2 - 2.0 * xc + c2

    def one_step(x, x2, c):
        d = dist(x, x2, c)
        z = -_TEMP * d
        z = z - jnp.max(z, axis=1, keepdims=True)
        e = jnp.exp(z)
        p = e / jnp.sum(e, axis=1, keepdims=True)
        w = p / (jnp.sum(p, axis=0, keepdims=True) + 1e-9)
        return lax.dot_general(
            w, x, (((0,), (0,)), ((), ())),
            preferred_element_type=jnp.float32,
            precision=lax.Precision.DEFAULT,
        )

    def step(_, cs):
        return tuple(one_step(x, x2, c) for x, x2, c in zip(xs, x2s, cs))

    cs = lax.fori_loop(0, _ITERS, step, tuple(x[:_L, :] for x in xs))

    li = lax.broadcasted_iota(jnp.int32, (_Q, _L), 1)
    for g, (x, x2, c) in enumerate(zip(xs, x2s, cs)):
        d = dist(x, x2, c)
        m = jnp.min(d, axis=1, keepdims=True)
        lab = jnp.min(jnp.where(d == m, li, _L), axis=1, keepdims=True)
        labels_ref[g] = lab + (pl.program_id(0) * _GPP + g) * _L
        centers_ref[g, :, : _D] = c


def _run_soft_kmeans(xr):
    return pl.pallas_call(
        _soft_kmeans_body,
        grid=(_R // _GPP,),
        in_specs=[pl.BlockSpec((_GPP, _Q, _D), lambda r: (r, 0, 0))],
        out_specs=[
            pl.BlockSpec((_GPP, _L, _DP), lambda r: (r, 0, 0)),
            pl.BlockSpec((_GPP, _Q, 1), lambda r: (r, 0, 0)),
        ],
        out_shape=[
            jax.ShapeDtypeStruct((_R, _L, _DP), jnp.float32),
            jax.ShapeDtypeStruct((_R, _Q, 1), jnp.int32),
        ],
        compiler_params=pltpu.CompilerParams(
            dimension_semantics=("parallel",),
        ),
    )(xr)


def _sc_gather(table, idx):
    info = plsc.get_sparse_core_info()
    nc, ns = info.num_cores, info.num_subcores
    nw = nc * ns
    b = _R * _Q
    b_per_w = b // nw
    mesh = plsc.VectorSubcoreMesh(core_axis_name="c", subcore_axis_name="s")

    @functools.partial(
        pl.kernel, mesh=mesh,
        out_type=jax.ShapeDtypeStruct((b, _DP), jnp.float32),
        scratch_types=[
            pltpu.VMEM((b_per_w,), jnp.int32),
            pltpu.VMEM((b_per_w, _DP), jnp.float32),
            pltpu.SemaphoreType.DMA,
        ],
    )
    def gather_k(table_hbm, idx_hbm, out_hbm, idx_v, rows_v, sem):
        wid = lax.axis_index("s") * nc + lax.axis_index("c")
        base = wid * b_per_w
        pltpu.sync_copy(idx_hbm.at[pl.ds(base, b_per_w)], idx_v)
        pltpu.async_copy(table_hbm.at[idx_v], rows_v, sem).wait()
        pltpu.sync_copy(rows_v, out_hbm.at[pl.ds(base, b_per_w)])

    return gather_k(table, idx)


def kernel(x):
    B, T, F = x.shape
    xr = x.reshape(_R, _Q, _D)
    centers, labels = _run_soft_kmeans(xr)
    table = centers.reshape(_R * _L, _DP)
    idx = labels.reshape(_R * _Q)
    rec = _sc_gather(table, idx)
    return rec[:, :_D].reshape(B, T, F)
